# baseline (speedup 1.0000x reference)
import functools

import jax
import jax.numpy as jnp
from jax import lax
from jax.experimental import pallas as pl
from jax.experimental.pallas import tpu as pltpu
from jax.experimental.pallas import tpu_sc as plsc

ALPHA = 0.2
ACT_DIM = 100000
BATCH = 128
MASK_LEN = 5000

NC = 2
NS = 16
NW = NC * NS

MTILE = 128
NTILE = 40
MPAD = NTILE * MTILE
FTILES_PER_W = 2

KSTEP = 512

_K0 = 0
_K1 = 42


def _sc_mesh():
    return plsc.VectorSubcoreMesh(core_axis_name="c", subcore_axis_name="s",
                                  num_cores=NC, num_subcores=NS)


def _sc_gather_body(qt_hbm, idx_hbm, qgt_hbm, ftbl_hbm,
                    idx_v, idxc_v, kval_v, dst_v, sem, sem2):
    wid = lax.axis_index("s") * NC + lax.axis_index("c")

    for t in range(FTILES_PER_W):
        j = wid + t * NW

        @pl.when(j < NTILE)
        def _fire():
            pltpu.sync_copy(idx_hbm.at[j], idx_v.at[t])
            for u in range(MTILE // 16):
                sl = pl.ds(u * 16, 16)
                idxc_v[t, sl] = jnp.minimum(idx_v[t, sl], ACT_DIM - 1)
                kval_v[t, sl] = lax.iota(jnp.int32, 16) + (j * MTILE + u * 16)
            pltpu.async_copy(qt_hbm.at[idxc_v.at[t]], dst_v.at[t], sem)
            pltpu.async_copy(kval_v.at[t], ftbl_hbm.at[idx_v.at[t]], sem2)

    for t in range(FTILES_PER_W):
        j = wid + t * NW

        @pl.when(j < NTILE)
        def _drain():
            pltpu.make_async_copy(qt_hbm.at[idxc_v.at[t]], dst_v.at[t],
                                  sem).wait()
            pltpu.sync_copy(dst_v.at[t], qgt_hbm.at[pl.ds(j * MTILE, MTILE)])
            pltpu.make_async_copy(kval_v.at[t], ftbl_hbm.at[idx_v.at[t]],
                                  sem2).wait()


def _sc_gather(qt, idx2):
    f = pl.kernel(
        _sc_gather_body,
        out_type=[
            jax.ShapeDtypeStruct((MPAD, BATCH), jnp.float32),
            jax.ShapeDtypeStruct((ACT_DIM + 8,), jnp.int32),
        ],
        mesh=_sc_mesh(),
        scratch_types=[
            pltpu.VMEM((FTILES_PER_W, MTILE), jnp.int32),
            pltpu.VMEM((FTILES_PER_W, MTILE), jnp.int32),
            pltpu.VMEM((FTILES_PER_W, MTILE), jnp.int32),
            pltpu.VMEM((FTILES_PER_W, MTILE, BATCH), jnp.float32),
            pltpu.SemaphoreType.DMA,
            pltpu.SemaphoreType.DMA,
        ],
    )
    return f(qt, idx2)


def _sc_flags_body(ftbl_hbm, idx_hbm, flg_hbm, idx_v, fl_v, sem):
    wid = lax.axis_index("s") * NC + lax.axis_index("c")
    for t in range(FTILES_PER_W):
        j = wid + t * NW

        @pl.when(j < NTILE)
        def _gb():
            pltpu.sync_copy(idx_hbm.at[j], idx_v.at[t])
            pltpu.async_copy(ftbl_hbm.at[idx_v.at[t]], fl_v.at[t], sem)

    for t in range(FTILES_PER_W):
        j = wid + t * NW

        @pl.when(j < NTILE)
        def _drain():
            pltpu.make_async_copy(ftbl_hbm.at[idx_v.at[t]], fl_v.at[t],
                                  sem).wait()
            pltpu.sync_copy(fl_v.at[t], flg_hbm.at[j])


def _sc_flags(ftbl, idx2):
    f = pl.kernel(
        _sc_flags_body,
        out_type=jax.ShapeDtypeStruct((NTILE, MTILE), jnp.int32),
        mesh=_sc_mesh(),
        scratch_types=[
            pltpu.VMEM((FTILES_PER_W, MTILE), jnp.int32),
            pltpu.VMEM((FTILES_PER_W, MTILE), jnp.int32),
            pltpu.SemaphoreType.DMA,
        ],
    )
    return f(ftbl, idx2)


def _rotl(x, r):
    return lax.shift_left(x, jnp.uint32(r)) | lax.shift_right_logical(
        x, jnp.uint32(32 - r))


def _threefry_bits(p):
    ks0 = jnp.uint32(_K0)
    ks1 = jnp.uint32(_K1)
    ks2 = ks0 ^ ks1 ^ jnp.uint32(0x1BD11BDA)
    rot1 = (13, 15, 26, 6)
    rot2 = (17, 29, 16, 24)
    x0 = jnp.zeros_like(p) + ks0
    x1 = p + ks1

    def rnds(x0, x1, rots):
        for r in rots:
            x0 = x0 + x1
            x1 = _rotl(x1, r)
            x1 = x1 ^ x0
        return x0, x1

    x0, x1 = rnds(x0, x1, rot1)
    x0 = x0 + ks1
    x1 = x1 + ks2 + jnp.uint32(1)
    x0, x1 = rnds(x0, x1, rot2)
    x0 = x0 + ks2
    x1 = x1 + ks0 + jnp.uint32(2)
    x0, x1 = rnds(x0, x1, rot1)
    x0 = x0 + ks0
    x1 = x1 + ks1 + jnp.uint32(3)
    x0, x1 = rnds(x0, x1, rot2)
    x0 = x0 + ks1
    x1 = x1 + ks2 + jnp.uint32(4)
    x0, x1 = rnds(x0, x1, rot1)
    x0 = x0 + ks2
    x1 = x1 + ks0 + jnp.uint32(5)
    return x0 ^ x1


def _gumbel_from_bits(bits):
    float_bits = lax.shift_right_logical(bits, jnp.uint32(9)) | jnp.uint32(
        0x3F800000)
    floats = lax.bitcast_convert_type(float_bits, jnp.float32) - jnp.float32(1.0)
    tiny = jnp.float32(jnp.finfo(jnp.float32).tiny)
    u = lax.max(tiny, floats * (jnp.float32(1.0) - tiny) + tiny)
    return -jnp.log(-jnp.log(u))


def _noise_body(idx_ref, g_ref):
    idx = idx_ref[...]
    row = lax.broadcasted_iota(jnp.int32, (KSTEP, BATCH), 1)
    p = (row * ACT_DIM + idx).astype(jnp.uint32)
    g_ref[...] = _gumbel_from_bits(_threefry_bits(p))


def _noise(idxc):
    grid = (MPAD // KSTEP,)
    return pl.pallas_call(
        _noise_body,
        grid=grid,
        in_specs=[pl.BlockSpec((KSTEP, 1), lambda i: (i, 0))],
        out_specs=pl.BlockSpec((KSTEP, BATCH), lambda i: (i, 0)),
        out_shape=jax.ShapeDtypeStruct((MPAD, BATCH), jnp.float32),
    )(idxc)


def _select_body(qgt_ref, g_ref, idx_ref, flg_ref, act_ref, logp_ref):
    qgt = qgt_ref[...]
    g = g_ref[...]
    idx = idx_ref[...]
    flg = flg_ref[...]

    kio = lax.broadcasted_iota(jnp.int32, (MPAD, 1), 0)
    valid = (kio < MASK_LEN) & (flg == kio)

    qs = qgt * jnp.float32(1.0 / ALPHA)
    neg = jnp.float32(-jnp.inf)
    qs_v = jnp.where(valid, qs, neg)

    m = jnp.max(qs_v, axis=0, keepdims=True)
    e = jnp.exp(qs_v - m)
    s = jnp.sum(jnp.where(valid, e, jnp.float32(0.0)), axis=0, keepdims=True)

    z = jnp.where(valid, qs + g, neg)
    zmax = jnp.max(z, axis=0, keepdims=True)
    big = jnp.int32(2**30)
    kstar = jnp.min(jnp.where(z >= zmax, kio, big), axis=0, keepdims=True)

    hit = kio == kstar
    act = jnp.max(jnp.where(hit, idx, jnp.int32(0)), axis=0, keepdims=True)
    esel = jnp.max(jnp.where(hit, e, jnp.float32(0.0)), axis=0, keepdims=True)

    act_ref[...] = act
    logp_ref[...] = esel / s


def _select(qgt, g, idxc, flags):
    return pl.pallas_call(
        _select_body,
        out_shape=[
            jax.ShapeDtypeStruct((1, BATCH), jnp.int32),
            jax.ShapeDtypeStruct((1, BATCH), jnp.float32),
        ],
    )(qgt, g, idxc, flags)


def kernel(q, action_mask):
    idx = action_mask.astype(jnp.int32)
    idxp = jnp.concatenate(
        [idx, jnp.full((MPAD - MASK_LEN,), ACT_DIM, jnp.int32)])
    idx2 = idxp.reshape(NTILE, MTILE)
    qt = jax.lax.optimization_barrier(q.T)

    g = _noise(idxp[:, None])
    qgt, ftbl = _sc_gather(qt, idx2)
    flg = _sc_flags(ftbl, idx2)

    act, logp = _select(qgt, g, idxp[:, None], flg.reshape(MPAD, 1))
    return act.reshape(BATCH, 1), logp.reshape(BATCH, 1)


# speedup vs baseline: 1.0041x; 1.0041x over previous
import functools

import jax
import jax.numpy as jnp
from jax import lax
from jax.experimental import pallas as pl
from jax.experimental.pallas import tpu as pltpu
from jax.experimental.pallas import tpu_sc as plsc

ALPHA = 0.2
ACT_DIM = 100000
BATCH = 128
MASK_LEN = 5000

NC = 2
NS = 16
NW = NC * NS

MTILE = 128
NTILE = 40
MPAD = NTILE * MTILE
FTILES_PER_W = 2

KSTEP = 512

_K0 = 0
_K1 = 42


def _sc_mesh():
    return plsc.VectorSubcoreMesh(core_axis_name="c", subcore_axis_name="s",
                                  num_cores=NC, num_subcores=NS)


def _sc_gather_body(qt_hbm, idx_hbm, qgt_hbm, ftbl_hbm,
                    idx_v, idxc_v, kval_v, dst_v, sem, sem2):
    wid = lax.axis_index("s") * NC + lax.axis_index("c")

    for t in range(FTILES_PER_W):
        j = wid + t * NW

        @pl.when(j < NTILE)
        def _fire():
            pltpu.sync_copy(idx_hbm.at[j], idx_v.at[t])
            for u in range(MTILE // 16):
                sl = pl.ds(u * 16, 16)
                idxc_v[t, sl] = jnp.minimum(idx_v[t, sl], ACT_DIM - 1)
                kval_v[t, sl] = lax.iota(jnp.int32, 16) + (j * MTILE + u * 16)
            pltpu.async_copy(qt_hbm.at[idxc_v.at[t]], dst_v.at[t], sem)
            pltpu.async_copy(kval_v.at[t], ftbl_hbm.at[idx_v.at[t]], sem2)

    for t in range(FTILES_PER_W):
        j = wid + t * NW

        @pl.when(j < NTILE)
        def _drain():
            pltpu.make_async_copy(qt_hbm.at[idxc_v.at[t]], dst_v.at[t],
                                  sem).wait()
            pltpu.sync_copy(dst_v.at[t], qgt_hbm.at[pl.ds(j * MTILE, MTILE)])
            pltpu.make_async_copy(kval_v.at[t], ftbl_hbm.at[idx_v.at[t]],
                                  sem2).wait()


def _sc_gather(qt, idx2):
    f = pl.kernel(
        _sc_gather_body,
        out_type=[
            jax.ShapeDtypeStruct((MPAD, BATCH), jnp.float32),
            jax.ShapeDtypeStruct((ACT_DIM + 8,), jnp.int32),
        ],
        mesh=_sc_mesh(),
        scratch_types=[
            pltpu.VMEM((FTILES_PER_W, MTILE), jnp.int32),
            pltpu.VMEM((FTILES_PER_W, MTILE), jnp.int32),
            pltpu.VMEM((FTILES_PER_W, MTILE), jnp.int32),
            pltpu.VMEM((FTILES_PER_W, MTILE, BATCH), jnp.float32),
            pltpu.SemaphoreType.DMA,
            pltpu.SemaphoreType.DMA,
        ],
    )
    return f(qt, idx2)


def _sc_flags_body(ftbl_hbm, idx_hbm, flg_hbm, idx_v, fl_v, sem):
    wid = lax.axis_index("s") * NC + lax.axis_index("c")
    for t in range(FTILES_PER_W):
        j = wid + t * NW

        @pl.when(j < NTILE)
        def _gb():
            pltpu.sync_copy(idx_hbm.at[j], idx_v.at[t])
            pltpu.async_copy(ftbl_hbm.at[idx_v.at[t]], fl_v.at[t], sem)

    for t in range(FTILES_PER_W):
        j = wid + t * NW

        @pl.when(j < NTILE)
        def _drain():
            pltpu.make_async_copy(ftbl_hbm.at[idx_v.at[t]], fl_v.at[t],
                                  sem).wait()
            pltpu.sync_copy(fl_v.at[t], flg_hbm.at[j])


def _sc_flags(ftbl, idx2):
    f = pl.kernel(
        _sc_flags_body,
        out_type=jax.ShapeDtypeStruct((NTILE, MTILE), jnp.int32),
        mesh=_sc_mesh(),
        scratch_types=[
            pltpu.VMEM((FTILES_PER_W, MTILE), jnp.int32),
            pltpu.VMEM((FTILES_PER_W, MTILE), jnp.int32),
            pltpu.SemaphoreType.DMA,
        ],
    )
    return f(ftbl, idx2)


def _rotl(x, r):
    return lax.shift_left(x, jnp.uint32(r)) | lax.shift_right_logical(
        x, jnp.uint32(32 - r))


def _threefry_bits(p):
    ks0 = jnp.uint32(_K0)
    ks1 = jnp.uint32(_K1)
    ks2 = ks0 ^ ks1 ^ jnp.uint32(0x1BD11BDA)
    rot1 = (13, 15, 26, 6)
    rot2 = (17, 29, 16, 24)
    x0 = jnp.zeros_like(p) + ks0
    x1 = p + ks1

    def rnds(x0, x1, rots):
        for r in rots:
            x0 = x0 + x1
            x1 = _rotl(x1, r)
            x1 = x1 ^ x0
        return x0, x1

    x0, x1 = rnds(x0, x1, rot1)
    x0 = x0 + ks1
    x1 = x1 + ks2 + jnp.uint32(1)
    x0, x1 = rnds(x0, x1, rot2)
    x0 = x0 + ks2
    x1 = x1 + ks0 + jnp.uint32(2)
    x0, x1 = rnds(x0, x1, rot1)
    x0 = x0 + ks0
    x1 = x1 + ks1 + jnp.uint32(3)
    x0, x1 = rnds(x0, x1, rot2)
    x0 = x0 + ks1
    x1 = x1 + ks2 + jnp.uint32(4)
    x0, x1 = rnds(x0, x1, rot1)
    x0 = x0 + ks2
    x1 = x1 + ks0 + jnp.uint32(5)
    return x0 ^ x1


def _gumbel_from_bits(bits):
    float_bits = lax.shift_right_logical(bits, jnp.uint32(9)) | jnp.uint32(
        0x3F800000)
    floats = lax.bitcast_convert_type(float_bits, jnp.float32) - jnp.float32(1.0)
    tiny = jnp.float32(jnp.finfo(jnp.float32).tiny)
    u = lax.max(tiny, floats * (jnp.float32(1.0) - tiny) + tiny)
    return -jnp.log(-jnp.log(u))


def _noise_body(idx_ref, g_ref):
    idx = idx_ref[...]
    row = lax.broadcasted_iota(jnp.int32, (KSTEP, BATCH), 1)
    p = (row * ACT_DIM + idx).astype(jnp.uint32)
    g_ref[...] = _gumbel_from_bits(_threefry_bits(p))


def _noise(idxc):
    grid = (MPAD // KSTEP,)
    return pl.pallas_call(
        _noise_body,
        grid=grid,
        in_specs=[pl.BlockSpec((KSTEP, 1), lambda i: (i, 0))],
        out_specs=pl.BlockSpec((KSTEP, BATCH), lambda i: (i, 0)),
        out_shape=jax.ShapeDtypeStruct((MPAD, BATCH), jnp.float32),
    )(idxc)


def _select_body(qgt_ref, g_ref, idx_ref, flg_ref, act_ref, logp_ref):
    qgt = qgt_ref[...]
    g = g_ref[...]
    idx = idx_ref[...]
    flg = flg_ref[...]

    kio = lax.broadcasted_iota(jnp.int32, (MPAD, 1), 0)
    valid = (kio < MASK_LEN) & (flg == kio)

    qs = qgt * jnp.float32(1.0 / ALPHA)
    neg = jnp.float32(-jnp.inf)
    qs_v = jnp.where(valid, qs, neg)

    m = jnp.max(qs_v, axis=0, keepdims=True)
    e = jnp.exp(qs_v - m)
    s = jnp.sum(jnp.where(valid, e, jnp.float32(0.0)), axis=0, keepdims=True)

    z = jnp.where(valid, qs + g, neg)
    zmax = jnp.max(z, axis=0, keepdims=True)
    big = jnp.int32(2**30)
    kstar = jnp.min(jnp.where(z >= zmax, kio, big), axis=0, keepdims=True)

    hit = kio == kstar
    act = jnp.max(jnp.where(hit, idx, jnp.int32(0)), axis=0, keepdims=True)
    esel = jnp.max(jnp.where(hit, e, jnp.float32(0.0)), axis=0, keepdims=True)

    act_ref[...] = act
    logp_ref[...] = esel / s


def _select(qgt, g, idxc, flags):
    return pl.pallas_call(
        _select_body,
        out_shape=[
            jax.ShapeDtypeStruct((1, BATCH), jnp.int32),
            jax.ShapeDtypeStruct((1, BATCH), jnp.float32),
        ],
    )(qgt, g, idxc, flags)


def kernel(q, action_mask):
    idx = action_mask.astype(jnp.int32)
    idxp = jnp.concatenate(
        [idx, jnp.full((MPAD - MASK_LEN,), ACT_DIM, jnp.int32)])
    idx2 = idxp.reshape(NTILE, MTILE)
    qt = q.T + jnp.float32(0.0)

    g = _noise(idxp[:, None])
    qgt, ftbl = _sc_gather(qt, idx2)
    flg = _sc_flags(ftbl, idx2)

    act, logp = _select(qgt, g, idxp[:, None], flg.reshape(MPAD, 1))
    return act.reshape(BATCH, 1), logp.reshape(BATCH, 1)
